# baseline (speedup 1.0000x reference)
import functools

import jax
import jax.numpy as jnp
from jax import lax
from jax.experimental import pallas as pl
from jax.experimental.pallas import tpu as pltpu
from jax.experimental.pallas import tpu_sc as plsc

_SEG = 2048
_HW = 196
_SND_CHUNK = 128


def _make_sc_kernel(N, C, n_seg, B):
    info = plsc.get_sparse_core_info()
    nc, ns = info.num_cores, info.num_subcores
    nw = nc * ns
    rows_w = N // nw
    nk = rows_w // _SND_CHUNK
    ng = C // 16
    segs_per_core = n_seg // nc
    mesh = plsc.VectorSubcoreMesh(core_axis_name="c", subcore_axis_name="s")

    @functools.partial(
        pl.kernel,
        out_type=jax.ShapeDtypeStruct((n_seg, B, C), jnp.float32),
        mesh=mesh,
        scratch_types=[
            pltpu.VMEM((2, _SND_CHUNK, C), jnp.float32),
            pltpu.VMEM((1, 1, C), jnp.float32),
            pltpu.VMEM((2, 1, C), jnp.float32),
            pltpu.VMEM((B, C), jnp.float32),
            pltpu.VMEM((16,), jnp.float32),
            pltpu.VMEM_SHARED((ns, 1, C), jnp.float32),
            pltpu.SemaphoreType.DMA,
            pltpu.SemaphoreType.DMA,
            pltpu.SemaphoreType.DMA,
        ],
    )
    def seg_sums(z_hbm, inv_hbm, msnd_hbm, buf, row_v, pair_v, blk_v, inv_v,
                 shared, sem0, sem1, sem2):
        cid = lax.axis_index("c")
        sid = lax.axis_index("s")
        wid = cid * ns + sid
        base = wid * rows_w
        sems = (sem0, sem1)

        def copy(k):
            return pltpu.make_async_copy(
                z_hbm.at[pl.ds(base + k * _SND_CHUNK, _SND_CHUNK), :],
                buf.at[k % 2], sems[k % 2])

        copy(0).start()
        pltpu.async_copy(inv_hbm, inv_v, sem2).wait()
        accs = tuple(jnp.zeros((16,), jnp.float32) for _ in range(ng))
        for k in range(nk):
            if k + 1 < nk:
                copy(k + 1).start()
            copy(k).wait()
            slot = buf.at[k % 2]

            def body(i, a, slot=slot):
                r = i * 8
                for u in range(8):
                    a = tuple(
                        a[c] + slot[r + u, c * 16:(c + 1) * 16]
                        for c in range(ng))
                return a

            accs = lax.fori_loop(0, _SND_CHUNK // 8, body, accs)
        for c in range(ng):
            row_v[0, 0, c * 16:(c + 1) * 16] = accs[c]

        pltpu.sync_copy(row_v, shared.at[pl.ds(sid, 1)])
        plsc.subcore_barrier()

        @pl.when(sid % 2 == 0)
        def _():
            pltpu.sync_copy(shared.at[pl.ds(sid, 2)], pair_v)
            inv = inv_v[0:16]
            for c in range(ng):
                sl = pl.ds(c * 16, 16)
                row_v[0, 0, sl] = (pair_v[0, 0, sl] + pair_v[1, 0, sl]) * inv
            for r in range(B):
                for c in range(ng):
                    sl = pl.ds(c * 16, 16)
                    blk_v[r, sl] = row_v[0, 0, sl]
            seg = cid * segs_per_core + sid // 2
            pltpu.sync_copy(blk_v, msnd_hbm.at[seg])

    return seg_sums


def _img_body(x_ref, mimg_ref):
    m = jnp.sum(x_ref[...], axis=2) * (1.0 / _HW)
    mimg_ref[...] = jnp.broadcast_to(m[None, :, :], mimg_ref.shape)


def kernel(Z_img, Z_snd, snd_splits):
    B, C, H, W = Z_img.shape
    N = Z_snd.shape[0]
    n_seg = N // _SEG

    inv = jnp.full((16,), 1.0, jnp.float32) / jnp.asarray(
        snd_splits).astype(jnp.float32)
    M_snd = _make_sc_kernel(N, C, n_seg, B)(Z_snd, inv)

    Z_img_flat = Z_img.reshape(B, C, H * W)
    M_img = pl.pallas_call(
        _img_body,
        grid=(B // 8,),
        in_specs=[pl.BlockSpec((8, C, H * W), lambda i: (i, 0, 0))],
        out_specs=pl.BlockSpec((n_seg, 8, C), lambda i: (0, i, 0)),
        out_shape=jax.ShapeDtypeStruct((n_seg, B, C), jnp.float32),
    )(Z_img_flat)
    return (M_img, M_snd)


# speedup vs baseline: 1.0608x; 1.0608x over previous
import functools

import jax
import jax.numpy as jnp
from jax import lax
from jax.experimental import pallas as pl
from jax.experimental.pallas import tpu as pltpu
from jax.experimental.pallas import tpu_sc as plsc

_SEG = 2048
_HW = 196
_SND_CHUNK = 128


def _make_sc_kernel(N, C, n_seg, B):
    info = plsc.get_sparse_core_info()
    nc, ns = info.num_cores, info.num_subcores
    nw = nc * ns
    rows_w = N // nw
    nk = rows_w // _SND_CHUNK
    ng = C // 16
    segs_per_core = n_seg // nc
    mesh = plsc.VectorSubcoreMesh(core_axis_name="c", subcore_axis_name="s")

    @functools.partial(
        pl.kernel,
        out_type=jax.ShapeDtypeStruct((n_seg, B, C), jnp.float32),
        mesh=mesh,
        scratch_types=[
            pltpu.VMEM((2, _SND_CHUNK, C), jnp.float32),
            pltpu.VMEM((1, 1, C), jnp.float32),
            pltpu.VMEM((2, 1, C), jnp.float32),
            pltpu.VMEM((B, C), jnp.float32),
            pltpu.VMEM((16,), jnp.float32),
            pltpu.VMEM_SHARED((ns, 1, C), jnp.float32),
            pltpu.SemaphoreType.DMA,
            pltpu.SemaphoreType.DMA,
            pltpu.SemaphoreType.DMA,
        ],
    )
    def seg_sums(z_hbm, inv_hbm, msnd_hbm, buf, row_v, pair_v, blk_v, inv_v,
                 shared, sem0, sem1, sem2):
        cid = lax.axis_index("c")
        sid = lax.axis_index("s")
        wid = cid * ns + sid
        base = wid * rows_w
        sems = (sem0, sem1)

        def copy(k):
            return pltpu.make_async_copy(
                z_hbm.at[pl.ds(base + k * _SND_CHUNK, _SND_CHUNK), :],
                buf.at[k % 2], sems[k % 2])

        copy(0).start()
        pltpu.async_copy(inv_hbm, inv_v, sem2).wait()
        accs = tuple(jnp.zeros((16,), jnp.float32) for _ in range(ng))
        for k in range(nk):
            if k + 1 < nk:
                copy(k + 1).start()
            copy(k).wait()
            slot = buf.at[k % 2]

            def body(i, a, slot=slot):
                r = i * 4
                for u in range(4):
                    a = tuple(
                        a[c] + slot[r + u, c * 16:(c + 1) * 16]
                        for c in range(ng))
                return a

            accs = lax.fori_loop(0, _SND_CHUNK // 4, body, accs)
        for c in range(ng):
            row_v[0, 0, c * 16:(c + 1) * 16] = accs[c]

        pltpu.sync_copy(row_v, shared.at[pl.ds(sid, 1)])
        plsc.subcore_barrier()

        @pl.when(sid % 2 == 0)
        def _():
            pltpu.sync_copy(shared.at[pl.ds(sid, 2)], pair_v)
            inv = inv_v[0:16]
            for c in range(ng):
                sl = pl.ds(c * 16, 16)
                row_v[0, 0, sl] = (pair_v[0, 0, sl] + pair_v[1, 0, sl]) * inv
            for r in range(B):
                for c in range(ng):
                    sl = pl.ds(c * 16, 16)
                    blk_v[r, sl] = row_v[0, 0, sl]
            seg = cid * segs_per_core + sid // 2
            pltpu.sync_copy(blk_v, msnd_hbm.at[seg])

    return seg_sums


def _img_body(x_ref, mimg_ref):
    m = jnp.sum(x_ref[...], axis=2) * (1.0 / _HW)
    mimg_ref[...] = jnp.broadcast_to(m[None, :, :], mimg_ref.shape)


def kernel(Z_img, Z_snd, snd_splits):
    B, C, H, W = Z_img.shape
    N = Z_snd.shape[0]
    n_seg = N // _SEG

    inv = jnp.full((16,), 1.0, jnp.float32) / jnp.asarray(
        snd_splits).astype(jnp.float32)
    M_snd = _make_sc_kernel(N, C, n_seg, B)(Z_snd, inv)

    Z_img_flat = Z_img.reshape(B, C, H * W)
    M_img = pl.pallas_call(
        _img_body,
        grid=(B // 8,),
        in_specs=[pl.BlockSpec((8, C, H * W), lambda i: (i, 0, 0))],
        out_specs=pl.BlockSpec((n_seg, 8, C), lambda i: (0, i, 0)),
        out_shape=jax.ShapeDtypeStruct((n_seg, B, C), jnp.float32),
    )(Z_img_flat)
    return (M_img, M_snd)
